# baseline (speedup 1.0000x reference)
import jax
import jax.numpy as jnp
from jax import lax
from jax.experimental import pallas as pl
from jax.experimental.pallas import tpu as pltpu


def _pos_kernel(row_ref, col_ref, out_ref, plane_ref, *, h, w, d):
    b = pl.program_id(0)
    hw = h * w

    @pl.when(b == 0)
    def _build_plane():
        k = lax.broadcasted_iota(jnp.int32, (h, hw), 1)
        r = lax.broadcasted_iota(jnp.int32, (h, hw), 0)
        sel_col = (k % w == r).astype(jnp.float32)
        sel_row = (k // w == r).astype(jnp.float32)
        col = col_ref[:w, :]
        row = row_ref[:h, :]
        plane_ref[:d, :] = lax.dot_general(
            col, sel_col, (((0,), (0,)), ((), ())),
            preferred_element_type=jnp.float32)
        plane_ref[d:, :] = lax.dot_general(
            row, sel_row, (((0,), (0,)), ((), ())),
            preferred_element_type=jnp.float32)

    out_ref[0, :, :] = plane_ref[:, :]


def kernel(x, row_weight, col_weight):
    b = x.shape[0]
    h, w = x.shape[-2], x.shape[-1]
    d = row_weight.shape[1]
    hw = h * w

    import functools
    body = functools.partial(_pos_kernel, h=h, w=w, d=d)
    out = pl.pallas_call(
        body,
        grid=(b,),
        in_specs=[
            pl.BlockSpec(row_weight.shape, lambda i: (0, 0)),
            pl.BlockSpec(col_weight.shape, lambda i: (0, 0)),
        ],
        out_specs=pl.BlockSpec((1, 2 * d, hw), lambda i: (i, 0, 0)),
        out_shape=jax.ShapeDtypeStruct((b, 2 * d, hw), jnp.float32),
        scratch_shapes=[pltpu.VMEM((2 * d, hw), jnp.float32)],
    )(row_weight, col_weight)
    return out.reshape(b, 2 * d, h, w)


# speedup vs baseline: 1.0885x; 1.0885x over previous
import functools

import jax
import jax.numpy as jnp
from jax import lax
from jax.experimental import pallas as pl
from jax.experimental.pallas import tpu as pltpu


def _pos_kernel(row_ref, col_ref, out_ref, plane_ref, sems, *, b, h, w, d):
    hw = h * w
    k = lax.broadcasted_iota(jnp.int32, (h, hw), 1)
    r = lax.broadcasted_iota(jnp.int32, (h, hw), 0)
    sel_col = (k % w == r).astype(jnp.float32)
    sel_row = (k // w == r).astype(jnp.float32)
    col = col_ref[:w, :]
    row = row_ref[:h, :]
    plane_ref[:d, :] = lax.dot_general(
        col, sel_col, (((0,), (0,)), ((), ())),
        preferred_element_type=jnp.float32,
        precision=lax.Precision.HIGHEST)
    plane_ref[d:, :] = lax.dot_general(
        row, sel_row, (((0,), (0,)), ((), ())),
        preferred_element_type=jnp.float32,
        precision=lax.Precision.HIGHEST)

    copies = [
        pltpu.make_async_copy(plane_ref, out_ref.at[i], sems.at[i])
        for i in range(b)
    ]
    for c in copies:
        c.start()
    for c in copies:
        c.wait()


def kernel(x, row_weight, col_weight):
    b = x.shape[0]
    h, w = x.shape[-2], x.shape[-1]
    d = row_weight.shape[1]
    hw = h * w

    body = functools.partial(_pos_kernel, b=b, h=h, w=w, d=d)
    out = pl.pallas_call(
        body,
        in_specs=[
            pl.BlockSpec(memory_space=pltpu.MemorySpace.VMEM),
            pl.BlockSpec(memory_space=pltpu.MemorySpace.VMEM),
        ],
        out_specs=pl.BlockSpec(memory_space=pltpu.MemorySpace.HBM),
        out_shape=jax.ShapeDtypeStruct((b, 2 * d, hw), jnp.float32),
        scratch_shapes=[
            pltpu.VMEM((2 * d, hw), jnp.float32),
            pltpu.SemaphoreType.DMA((b,)),
        ],
    )(row_weight, col_weight)
    return out.reshape(b, 2 * d, h, w)
